# baseline (speedup 1.0000x reference)
import functools

import jax
import jax.numpy as jnp
from jax import lax
from jax.experimental import pallas as pl
from jax.experimental.pallas import tpu as pltpu
from jax.experimental.pallas import tpu_sc as plsc

DIM = 128
HALF_DIM = 64
NUM_EMB = 1000
BATCH = 16384

NC = 2
NS = 16
NW = NC * NS
B_PER_W = BATCH // NW
CHUNK = 64
N_CHUNKS = B_PER_W // CHUNK


def _fuse_table_body(embt_ref, wt_ref, b_ref, out_ref):
    e = embt_ref[...]
    e = e * jax.nn.sigmoid(e)
    acc = lax.dot_general(e, wt_ref[...], (((0,), (0,)), ((), ())),
                          preferred_element_type=jnp.float32)
    out_ref[...] = acc + b_ref[...]


def _fuse_table(emb_t, W_t, b2d):
    return pl.pallas_call(
        _fuse_table_body,
        grid=(4,),
        in_specs=[
            pl.BlockSpec((HALF_DIM, 256), lambda i: (0, i)),
            pl.BlockSpec((HALF_DIM, DIM), lambda i: (0, 0)),
            pl.BlockSpec((1, DIM), lambda i: (0, 0)),
        ],
        out_specs=pl.BlockSpec((256, DIM), lambda i: (i, 0)),
        out_shape=jax.ShapeDtypeStruct((NUM_EMB, DIM), jnp.float32),
    )(emb_t, W_t, b2d)


@functools.cache
def _make_gather_kernel():
    mesh = plsc.VectorSubcoreMesh(core_axis_name="c", subcore_axis_name="s")

    @functools.partial(
        pl.kernel,
        mesh=mesh,
        out_type=jax.ShapeDtypeStruct((BATCH, DIM), jnp.float32),
        scratch_types=[
            pltpu.VMEM((B_PER_W,), jnp.int32),
            pltpu.VMEM((B_PER_W, DIM), jnp.float32),
            pltpu.VMEM_SHARED((NUM_EMB, DIM), jnp.float32),
        ]
        + [pltpu.SemaphoreType.DMA] * (N_CHUNKS + 2),
    )
    def _gather_kernel(table_hbm, idx_hbm, out_hbm, idx_v, rows_v, tab_sp, *sems):
        isem, gsem, wsems = sems[0], sems[1], sems[2:]
        sid = lax.axis_index("s")
        wid = sid * NC + lax.axis_index("c")
        base = wid * B_PER_W
        idx_copy = pltpu.async_copy(idx_hbm.at[pl.ds(base, B_PER_W)], idx_v, isem)
        @pl.when(sid < 15)
        def _stage():
            pltpu.sync_copy(
                table_hbm.at[pl.ds(sid * 64, 64)],
                tab_sp.at[pl.ds(sid * 64, 64)],
            )
        @pl.when(sid == 15)
        def _stage_tail():
            pltpu.sync_copy(
                table_hbm.at[pl.ds(960, NUM_EMB - 960)],
                tab_sp.at[pl.ds(960, NUM_EMB - 960)],
            )
        idx_copy.wait()
        plsc.subcore_barrier()
        gathers = [
            pltpu.async_copy(
                tab_sp.at[idx_v.at[pl.ds(j * CHUNK, CHUNK)]],
                rows_v.at[pl.ds(j * CHUNK, CHUNK)],
                gsem,
            )
            for j in range(N_CHUNKS)
        ]
        writes = []
        for j in range(N_CHUNKS):
            gathers[j].wait()
            writes.append(
                pltpu.async_copy(
                    rows_v.at[pl.ds(j * CHUNK, CHUNK)],
                    out_hbm.at[pl.ds(base + j * CHUNK, CHUNK)],
                    wsems[j],
                )
            )
        for w in writes:
            w.wait()

    return _gather_kernel


def kernel(t, emb_table, W, b):
    t32 = t.astype(jnp.int32)
    fused = _fuse_table(emb_table.T, W.T, b.reshape(1, DIM))
    return _make_gather_kernel()(fused, t32)


# speedup vs baseline: 1.0269x; 1.0269x over previous
import functools

import jax
import jax.numpy as jnp
from jax import lax
from jax.experimental import pallas as pl
from jax.experimental.pallas import tpu as pltpu
from jax.experimental.pallas import tpu_sc as plsc

DIM = 128
HALF_DIM = 64
NUM_EMB = 1000
BATCH = 16384

NC = 2
NS = 16
NW = NC * NS
B_PER_W = BATCH // NW
CHUNK = 64
N_CHUNKS = B_PER_W // CHUNK


def _fuse_table_body(embt_ref, wt_ref, b_ref, out_ref):
    e = embt_ref[...]
    e = e * jax.nn.sigmoid(e)
    acc = lax.dot_general(e, wt_ref[...], (((0,), (0,)), ((), ())),
                          preferred_element_type=jnp.float32)
    out_ref[...] = acc + b_ref[...]


def _fuse_table(emb_t, W_t, b2d):
    return pl.pallas_call(
        _fuse_table_body,
        out_shape=jax.ShapeDtypeStruct((NUM_EMB, DIM), jnp.float32),
    )(emb_t, W_t, b2d)


@functools.cache
def _make_gather_kernel():
    mesh = plsc.VectorSubcoreMesh(core_axis_name="c", subcore_axis_name="s")

    @functools.partial(
        pl.kernel,
        mesh=mesh,
        out_type=jax.ShapeDtypeStruct((BATCH, DIM), jnp.float32),
        scratch_types=[
            pltpu.VMEM((B_PER_W,), jnp.int32),
            pltpu.VMEM((B_PER_W, DIM), jnp.float32),
            pltpu.VMEM_SHARED((NUM_EMB, DIM), jnp.float32),
        ]
        + [pltpu.SemaphoreType.DMA] * (N_CHUNKS + 2),
    )
    def _gather_kernel(table_hbm, idx_hbm, out_hbm, idx_v, rows_v, tab_sp, *sems):
        isem, gsem, wsems = sems[0], sems[1], sems[2:]
        sid = lax.axis_index("s")
        wid = sid * NC + lax.axis_index("c")
        base = wid * B_PER_W
        idx_copy = pltpu.async_copy(idx_hbm.at[pl.ds(base, B_PER_W)], idx_v, isem)
        @pl.when(sid < 15)
        def _stage():
            pltpu.sync_copy(
                table_hbm.at[pl.ds(sid * 64, 64)],
                tab_sp.at[pl.ds(sid * 64, 64)],
            )
        @pl.when(sid == 15)
        def _stage_tail():
            pltpu.sync_copy(
                table_hbm.at[pl.ds(960, NUM_EMB - 960)],
                tab_sp.at[pl.ds(960, NUM_EMB - 960)],
            )
        idx_copy.wait()
        plsc.subcore_barrier()
        gathers = [
            pltpu.async_copy(
                tab_sp.at[idx_v.at[pl.ds(j * CHUNK, CHUNK)]],
                rows_v.at[pl.ds(j * CHUNK, CHUNK)],
                gsem,
            )
            for j in range(N_CHUNKS)
        ]
        writes = []
        for j in range(N_CHUNKS):
            gathers[j].wait()
            writes.append(
                pltpu.async_copy(
                    rows_v.at[pl.ds(j * CHUNK, CHUNK)],
                    out_hbm.at[pl.ds(base + j * CHUNK, CHUNK)],
                    wsems[j],
                )
            )
        for w in writes:
            w.wait()

    return _gather_kernel


def kernel(t, emb_table, W, b):
    t32 = t.astype(jnp.int32)
    fused = _fuse_table(emb_table.T, W.T, b.reshape(1, DIM))
    return _make_gather_kernel()(fused, t32)
